# baseline (speedup 1.0000x reference)
import jax
import jax.numpy as jnp
from jax.experimental import pallas as pl
from jax.experimental.pallas import tpu as pltpu


def _round_up(v, m):
    return -(-v // m) * m


def _make_kernel(n_streams):
    def _body(*refs):
        x_refs = refs[:n_streams]
        o_refs = refs[n_streams:]
        for x_ref, o_ref in zip(x_refs, o_refs):
            m = jnp.max(x_ref[...], axis=1)
            o_ref[...] = jnp.max(m, axis=-1, keepdims=True)

    return _body


def kernel(x):
    N, C, H, W = x.shape
    dtype = x.dtype
    itemsize = jnp.dtype(dtype).itemsize

    rows = N * C
    xr = x.reshape(rows, H, W)

    K = 16
    RB = 32

    n_blocks = rows // RB
    steps = n_blocks // K
    grid = (steps,)
    rows_per_stream = rows // K

    in_block = RB * _round_up(H, 8) * _round_up(W, 128) * itemsize
    out_block = _round_up(RB, 8) * 128 * itemsize
    vmem_limit = max(
        4 << 20, min(K * (2 * in_block + 2 * out_block) + (2 << 20), 56 << 20)
    )

    cost = pl.CostEstimate(
        flops=rows * H * W,
        transcendentals=0,
        bytes_accessed=rows * H * W * itemsize + rows * itemsize,
    )

    def in_map(j):
        return lambda i: (j * steps + i, 0, 0)

    def out_map():
        return lambda i: (i, 0)

    in_specs = [pl.BlockSpec((RB, H, W), in_map(j)) for j in range(K)]
    out_specs = [pl.BlockSpec((RB, 1), out_map()) for _ in range(K)]
    out_shapes = [
        jax.ShapeDtypeStruct((rows_per_stream, 1), dtype) for _ in range(K)
    ]

    outs = pl.pallas_call(
        _make_kernel(K),
        out_shape=out_shapes,
        grid=grid,
        in_specs=in_specs,
        out_specs=out_specs,
        compiler_params=pltpu.CompilerParams(
            dimension_semantics=("parallel",),
            vmem_limit_bytes=int(vmem_limit),
        ),
        cost_estimate=cost,
    )(*([xr] * K))

    return jnp.concatenate(outs, axis=0).reshape(N, C)


# speedup vs baseline: 1.0095x; 1.0095x over previous
import jax
import jax.numpy as jnp
from jax.experimental import pallas as pl
from jax.experimental.pallas import tpu as pltpu


def _round_up(v, m):
    return -(-v // m) * m


def _make_kernel(n_streams):
    def _body(*refs):
        x_refs = refs[:n_streams]
        o_refs = refs[n_streams:]
        for x_ref, o_ref in zip(x_refs, o_refs):
            m = jnp.max(x_ref[...], axis=1)
            o_ref[...] = jnp.max(m, axis=-1, keepdims=True)

    return _body


def kernel(x):
    N, C, H, W = x.shape
    dtype = x.dtype
    itemsize = jnp.dtype(dtype).itemsize

    rows = N * C
    xr = x.reshape(rows, H, W)

    K = 2
    RB = 256

    n_blocks = rows // RB
    steps = n_blocks // K
    grid = (steps,)
    rows_per_stream = rows // K

    in_block = RB * _round_up(H, 8) * _round_up(W, 128) * itemsize
    out_block = _round_up(RB, 8) * 128 * itemsize
    vmem_limit = max(
        4 << 20, min(K * (2 * in_block + 2 * out_block) + (2 << 20), 56 << 20)
    )

    cost = pl.CostEstimate(
        flops=rows * H * W,
        transcendentals=0,
        bytes_accessed=rows * H * W * itemsize + rows * itemsize,
    )

    def in_map(j):
        return lambda i: (j * steps + i, 0, 0)

    def out_map():
        return lambda i: (i, 0)

    in_specs = [pl.BlockSpec((RB, H, W), in_map(j)) for j in range(K)]
    out_specs = [pl.BlockSpec((RB, 1), out_map()) for _ in range(K)]
    out_shapes = [
        jax.ShapeDtypeStruct((rows_per_stream, 1), dtype) for _ in range(K)
    ]

    outs = pl.pallas_call(
        _make_kernel(K),
        out_shape=out_shapes,
        grid=grid,
        in_specs=in_specs,
        out_specs=out_specs,
        compiler_params=pltpu.CompilerParams(
            dimension_semantics=("parallel",),
            vmem_limit_bytes=int(vmem_limit),
        ),
        cost_estimate=cost,
    )(*([xr] * K))

    return jnp.concatenate(outs, axis=0).reshape(N, C)


# speedup vs baseline: 4.8334x; 4.7880x over previous
import jax
import jax.numpy as jnp
from jax.experimental import pallas as pl
from jax.experimental.pallas import tpu as pltpu


def _make_body(n_streams):
    def _body(*refs):
        x_refs = refs[:n_streams]
        o_refs = refs[n_streams:]
        for x_ref, o_ref in zip(x_refs, o_refs):
            o_ref[...] = jnp.max(x_ref[0], axis=0)[None, None, :]

    return _body


def kernel(x):
    N, C, H, W = x.shape
    dtype = x.dtype
    itemsize = jnp.dtype(dtype).itemsize

    xt = jnp.transpose(x, (0, 2, 3, 1)).reshape(N, H * W, C)

    K = 2
    steps = N // K
    grid = (steps,)

    in_block = H * W * -(-C // 128) * 128 * itemsize
    vmem_limit = max(4 << 20, min(K * 2 * in_block + (2 << 20), 56 << 20))

    cost = pl.CostEstimate(
        flops=N * C * H * W,
        transcendentals=0,
        bytes_accessed=N * C * H * W * itemsize + N * C * itemsize,
    )

    in_specs = [
        pl.BlockSpec((1, H * W, C), (lambda i, j=j: (j * steps + i, 0, 0)))
        for j in range(K)
    ]
    out_specs = [pl.BlockSpec((1, 1, C), lambda i: (i, 0, 0)) for _ in range(K)]
    out_shapes = [jax.ShapeDtypeStruct((steps, 1, C), dtype) for _ in range(K)]

    outs = pl.pallas_call(
        _make_body(K),
        out_shape=out_shapes,
        grid=grid,
        in_specs=in_specs,
        out_specs=out_specs,
        compiler_params=pltpu.CompilerParams(
            dimension_semantics=("parallel",),
            vmem_limit_bytes=int(vmem_limit),
        ),
        cost_estimate=cost,
    )(*([xt] * K))

    return jnp.concatenate(outs, axis=0).reshape(N, C)
